# baseline (speedup 1.0000x reference)
import functools

import jax
import jax.numpy as jnp
from jax import lax
from jax.experimental import pallas as pl
from jax.experimental.pallas import tpu as pltpu

_CLASSES = 100


def _stat_kernel(xg_ref, xv_ref, w1_ref, w2_ref, b2_ref, o_ref, x_scr,
                 *, dense, feat, k_pad):
    bm = xg_ref.shape[0]
    x_scr[:, 0:feat] = xg_ref[...]
    x_scr[:, feat:2 * feat] = xv_ref[...]
    tail = lax.broadcasted_iota(jnp.int32, (bm, k_pad - 2 * feat), 1)
    x_scr[:, 2 * feat:] = jnp.where(tail == 0, 1.0, 0.0).astype(jnp.float32)

    y = jnp.dot(x_scr[...], w1_ref[...], preferred_element_type=jnp.float32)
    c_logits = y[:, :dense]
    f = y[:, dense:]

    c_m = jnp.max(c_logits, axis=-1, keepdims=True)
    c_e = jnp.exp(c_logits - c_m)
    c = c_e / jnp.sum(c_e, axis=-1, keepdims=True)

    dd = jnp.sum(c * f, axis=-1, keepdims=True)
    aln = (f - dd) / dd

    logits = jnp.dot(aln, w2_ref[...], preferred_element_type=jnp.float32) + b2_ref[...]
    m = jnp.max(logits, axis=-1, keepdims=True)
    e = jnp.exp(logits - m)
    p = e / jnp.sum(e, axis=-1, keepdims=True)
    o_ref[...] = p[:, :o_ref.shape[1]]


def kernel(f_gap, f_gvp, w1, w2, b2):
    B, F = f_gap.shape
    k_pad, two_dense = w1.shape
    dense = two_dense // 2
    n_pad = w2.shape[1]

    bm = min(512, B)
    grid = (B // bm,)

    out = pl.pallas_call(
        functools.partial(_stat_kernel, dense=dense, feat=F, k_pad=k_pad),
        out_shape=jax.ShapeDtypeStruct((B, _CLASSES), jnp.float32),
        grid=grid,
        in_specs=[
            pl.BlockSpec((bm, F), lambda i: (i, 0)),
            pl.BlockSpec((bm, F), lambda i: (i, 0)),
            pl.BlockSpec((k_pad, two_dense), lambda i: (0, 0)),
            pl.BlockSpec((dense, n_pad), lambda i: (0, 0)),
            pl.BlockSpec((1, n_pad), lambda i: (0, 0)),
        ],
        out_specs=pl.BlockSpec((bm, _CLASSES), lambda i: (i, 0)),
        scratch_shapes=[pltpu.VMEM((bm, k_pad), jnp.float32)],
        compiler_params=pltpu.CompilerParams(
            dimension_semantics=("parallel",)),
    )(f_gap.astype(jnp.float32), f_gvp.astype(jnp.float32), w1, w2, b2)

    return out


# speedup vs baseline: 1.2146x; 1.2146x over previous
import functools

import jax
import jax.numpy as jnp
from jax import lax
from jax.experimental import pallas as pl
from jax.experimental.pallas import tpu as pltpu

_CLASSES = 100


def _stat_kernel(xg_ref, xv_ref, w1_ref, w2_ref, b2_ref, o_ref, x_scr,
                 *, dense, feat, k_pad):
    bm = xg_ref.shape[0]
    x_scr[:, 0:feat] = xg_ref[...]
    x_scr[:, feat:2 * feat] = xv_ref[...]
    tail = lax.broadcasted_iota(jnp.int32, (bm, k_pad - 2 * feat), 1)
    x_scr[:, 2 * feat:] = jnp.where(tail == 0, 1.0, 0.0).astype(jnp.float32)

    y = jnp.dot(x_scr[...], w1_ref[...], preferred_element_type=jnp.float32)
    c_logits = y[:, :dense]
    f = y[:, dense:]

    c_m = jnp.max(c_logits, axis=-1, keepdims=True)
    c_e = jnp.exp(c_logits - c_m)
    c = c_e / jnp.sum(c_e, axis=-1, keepdims=True)

    dd = jnp.sum(c * f, axis=-1, keepdims=True)
    aln = (f - dd) / dd

    logits = jnp.dot(aln, w2_ref[...], preferred_element_type=jnp.float32) + b2_ref[...]
    m = jnp.max(logits, axis=-1, keepdims=True)
    e = jnp.exp(logits - m)
    p = e / jnp.sum(e, axis=-1, keepdims=True)
    o_ref[...] = p[:, :o_ref.shape[1]]


def kernel(f_gap, f_gvp, w1, w2, b2):
    B, F = f_gap.shape
    k_pad, two_dense = w1.shape
    dense = two_dense // 2
    n_pad = w2.shape[1]

    bm = min(1024, B)
    grid = (B // bm,)

    out = pl.pallas_call(
        functools.partial(_stat_kernel, dense=dense, feat=F, k_pad=k_pad),
        out_shape=jax.ShapeDtypeStruct((B, _CLASSES), jnp.float32),
        grid=grid,
        in_specs=[
            pl.BlockSpec((bm, F), lambda i: (i, 0)),
            pl.BlockSpec((bm, F), lambda i: (i, 0)),
            pl.BlockSpec((k_pad, two_dense), lambda i: (0, 0)),
            pl.BlockSpec((dense, n_pad), lambda i: (0, 0)),
            pl.BlockSpec((1, n_pad), lambda i: (0, 0)),
        ],
        out_specs=pl.BlockSpec((bm, _CLASSES), lambda i: (i, 0)),
        scratch_shapes=[pltpu.VMEM((bm, k_pad), jnp.float32)],
        compiler_params=pltpu.CompilerParams(
            dimension_semantics=("parallel",)),
    )(f_gap.astype(jnp.float32), f_gvp.astype(jnp.float32), w1, w2, b2)

    return out


# speedup vs baseline: 1.2153x; 1.0006x over previous
import functools

import jax
import jax.numpy as jnp
from jax import lax
from jax.experimental import pallas as pl
from jax.experimental.pallas import tpu as pltpu

_CLASSES = 100


def _stat_kernel(xg_ref, xv_ref, w1_ref, w2_ref, b2_ref, o_ref, x_scr,
                 *, dense, feat, k_pad):
    bm = xg_ref.shape[0]
    x_scr[:, 0:feat] = xg_ref[...]
    x_scr[:, feat:2 * feat] = xv_ref[...]
    tail = lax.broadcasted_iota(jnp.int32, (bm, k_pad - 2 * feat), 1)
    x_scr[:, 2 * feat:] = jnp.where(tail == 0, 1.0, 0.0).astype(jnp.float32)

    y = jnp.dot(x_scr[...], w1_ref[...], preferred_element_type=jnp.float32)
    c_logits = y[:, :dense]
    f = y[:, dense:]

    c_m = jnp.max(c_logits, axis=-1, keepdims=True)
    c_e = jnp.exp(c_logits - c_m)
    c = c_e / jnp.sum(c_e, axis=-1, keepdims=True)

    dd = jnp.sum(c * f, axis=-1, keepdims=True)
    aln = (f - dd) / dd

    logits = jnp.dot(aln, w2_ref[...], preferred_element_type=jnp.float32) + b2_ref[...]
    m = jnp.max(logits, axis=-1, keepdims=True)
    e = jnp.exp(logits - m)
    p = e / jnp.sum(e, axis=-1, keepdims=True)
    o_ref[...] = p[:, :o_ref.shape[1]]


def kernel(f_gap, f_gvp, w1, w2, b2):
    B, F = f_gap.shape
    k_pad, two_dense = w1.shape
    dense = two_dense // 2
    n_pad = w2.shape[1]

    bm = min(2048, B)
    grid = (B // bm,)

    out = pl.pallas_call(
        functools.partial(_stat_kernel, dense=dense, feat=F, k_pad=k_pad),
        out_shape=jax.ShapeDtypeStruct((B, _CLASSES), jnp.float32),
        grid=grid,
        in_specs=[
            pl.BlockSpec((bm, F), lambda i: (i, 0)),
            pl.BlockSpec((bm, F), lambda i: (i, 0)),
            pl.BlockSpec((k_pad, two_dense), lambda i: (0, 0)),
            pl.BlockSpec((dense, n_pad), lambda i: (0, 0)),
            pl.BlockSpec((1, n_pad), lambda i: (0, 0)),
        ],
        out_specs=pl.BlockSpec((bm, _CLASSES), lambda i: (i, 0)),
        scratch_shapes=[pltpu.VMEM((bm, k_pad), jnp.float32)],
        compiler_params=pltpu.CompilerParams(
            dimension_semantics=("parallel",)),
    )(f_gap.astype(jnp.float32), f_gvp.astype(jnp.float32), w1, w2, b2)

    return out


# speedup vs baseline: 1.2182x; 1.0024x over previous
import functools

import jax
import jax.numpy as jnp
from jax import lax
from jax.experimental import pallas as pl
from jax.experimental.pallas import tpu as pltpu

_CLASSES = 100


def _stat_kernel(xg_ref, xv_ref, w1_ref, w2_ref, b2_ref, o_ref, x_scr,
                 *, dense, feat, k_pad):
    bm = xg_ref.shape[0]
    x_scr[:, 0:feat] = xg_ref[...]
    x_scr[:, feat:2 * feat] = xv_ref[...]
    tail = lax.broadcasted_iota(jnp.int32, (bm, k_pad - 2 * feat), 1)
    x_scr[:, 2 * feat:] = jnp.where(tail == 0, 1.0, 0.0).astype(jnp.float32)

    y = jnp.dot(x_scr[...], w1_ref[...], preferred_element_type=jnp.float32)
    c_logits = y[:, :dense]
    f = y[:, dense:]

    c_m = jnp.max(c_logits, axis=-1, keepdims=True)
    c_e = jnp.exp(c_logits - c_m)
    c = c_e / jnp.sum(c_e, axis=-1, keepdims=True)

    dd = jnp.sum(c * f, axis=-1, keepdims=True)
    aln = (f - dd) / dd

    logits = jnp.dot(aln, w2_ref[...], preferred_element_type=jnp.float32) + b2_ref[...]
    m = jnp.max(logits, axis=-1, keepdims=True)
    e = jnp.exp(logits - m)
    p = e / jnp.sum(e, axis=-1, keepdims=True)
    o_ref[...] = p[:, :o_ref.shape[1]]


def kernel(f_gap, f_gvp, w1, w2, b2):
    B, F = f_gap.shape
    k_pad, two_dense = w1.shape
    dense = two_dense // 2
    n_pad = w2.shape[1]

    bm = min(1024, B)
    grid = (B // bm,)

    out = pl.pallas_call(
        functools.partial(_stat_kernel, dense=dense, feat=F, k_pad=k_pad),
        out_shape=jax.ShapeDtypeStruct((B, _CLASSES), jnp.float32),
        grid=grid,
        in_specs=[
            pl.BlockSpec((bm, F), lambda i: (i, 0)),
            pl.BlockSpec((bm, F), lambda i: (i, 0)),
            pl.BlockSpec((k_pad, two_dense), lambda i: (0, 0)),
            pl.BlockSpec((dense, n_pad), lambda i: (0, 0)),
            pl.BlockSpec((1, n_pad), lambda i: (0, 0)),
        ],
        out_specs=pl.BlockSpec((bm, _CLASSES), lambda i: (i, 0)),
        scratch_shapes=[pltpu.VMEM((bm, k_pad), jnp.float32)],
        compiler_params=pltpu.CompilerParams(
            dimension_semantics=("arbitrary",)),
    )(f_gap.astype(jnp.float32), f_gvp.astype(jnp.float32), w1, w2, b2)

    return out
